# baseline (speedup 1.0000x reference)
import jax
import jax.numpy as jnp
from jax import lax
from jax.experimental import pallas as pl
from jax.experimental.pallas import tpu as pltpu
from jax.experimental.pallas import tpu_sc as plsc

B, S, D = 32, 512, 768
T = B * S
NC, NS = 2, 16
NW = NC * NS
NCHUNK = 4
CT = T // NCHUNK
TOK_PER_W = CT // NW
F = 64
NSUB = TOK_PER_W // F
BT = 512
EPS = 1e-12


def _sc_gather_body(table_hbm, ids_hbm, out_hbm, idx_v, rows_v, sem):
    wid = lax.axis_index("s") * NC + lax.axis_index("c")
    base = wid * TOK_PER_W
    for f in range(NSUB):
        off = base + f * F
        pltpu.sync_copy(ids_hbm.at[pl.ds(off, F)], idx_v)
        pltpu.async_copy(table_hbm.at[idx_v], rows_v, sem).wait()
        pltpu.sync_copy(rows_v, out_hbm.at[pl.ds(off, F)])


_sc_gather = pl.kernel(
    _sc_gather_body,
    out_type=jax.ShapeDtypeStruct((CT, D), jnp.float32),
    mesh=plsc.VectorSubcoreMesh(
        core_axis_name="c", subcore_axis_name="s", num_cores=NC, num_subcores=NS
    ),
    scratch_types=[
        pltpu.VMEM((F,), jnp.int32),
        pltpu.VMEM((F, D), jnp.float32),
        pltpu.SemaphoreType.DMA,
    ],
)


def _make_tc_ln(chunk, aliased):
    nblk = CT // BT

    def body(*refs):
        if aliased:
            _, g_ref, tts_ref, pos_ref, wt_ref, gamma_ref, beta_ref, out_ref, y_v, sem = refs
        else:
            g_ref, tts_ref, pos_ref, wt_ref, gamma_ref, beta_ref, out_ref, y_v, sem = refs
        i = pl.program_id(0)
        x = g_ref[...]
        t = tts_ref[...]
        w0 = wt_ref[0:1, :]
        diff = wt_ref[1:2, :] - w0
        x = x + pos_ref[...] + w0 + t * diff
        s1 = jnp.sum(x, axis=-1, keepdims=True)
        s2 = jnp.sum(x * x, axis=-1, keepdims=True)
        mean = s1 * (1.0 / D)
        var = jnp.maximum(s2 * (1.0 / D) - mean * mean, 0.0)
        rstd = lax.rsqrt(var + EPS)
        y_v[...] = (x - mean) * rstd * gamma_ref[...] + beta_ref[...]
        pltpu.sync_copy(y_v, out_ref.at[pl.ds((chunk * nblk + i) * BT, BT)])

    in_specs = [
        pl.BlockSpec((BT, D), lambda i: (i, 0)),
        pl.BlockSpec((BT, 1), lambda i: (i, 0)),
        pl.BlockSpec((S, D), lambda i: (0, 0)),
        pl.BlockSpec((2, D), lambda i: (0, 0)),
        pl.BlockSpec((1, D), lambda i: (0, 0)),
        pl.BlockSpec((1, D), lambda i: (0, 0)),
    ]
    kwargs = {}
    if aliased:
        in_specs = [pl.BlockSpec(memory_space=pl.ANY)] + in_specs
        kwargs["input_output_aliases"] = {0: 0}
    return pl.pallas_call(
        body,
        grid=(nblk,),
        in_specs=in_specs,
        out_specs=pl.BlockSpec(memory_space=pl.ANY),
        out_shape=jax.ShapeDtypeStruct((T, D), jnp.float32),
        scratch_shapes=[pltpu.VMEM((BT, D), jnp.float32), pltpu.SemaphoreType.DMA],
        **kwargs,
    )


def kernel(input_ids, token_type_ids, W_word, W_pos, W_type, gamma, beta):
    ids_flat = input_ids.reshape(T).astype(jnp.int32)
    tts = token_type_ids.reshape(T, 1).astype(jnp.float32)
    gamma2 = gamma.reshape(1, D)
    beta2 = beta.reshape(1, D)

    gathered = [
        _sc_gather(W_word, ids_flat[c * CT:(c + 1) * CT]) for c in range(NCHUNK)
    ]

    out = None
    for c in range(NCHUNK):
        args = (gathered[c], tts[c * CT:(c + 1) * CT], W_pos, W_type, gamma2, beta2)
        if c == 0:
            out = _make_tc_ln(c, aliased=False)(*args)
        else:
            out = _make_tc_ln(c, aliased=True)(out, *args)

    return out.reshape(B, S, D)


# speedup vs baseline: 1.1290x; 1.1290x over previous
import jax
import jax.numpy as jnp
from jax import lax
from jax.experimental import pallas as pl
from jax.experimental.pallas import tpu as pltpu
from jax.experimental.pallas import tpu_sc as plsc

B, S, D = 32, 512, 768
T = B * S
NC, NS = 2, 16
NW = NC * NS
NCHUNK = 4
CT = T // NCHUNK
TOK_PER_W = CT // NW
F = 64
NSUB = TOK_PER_W // F
BT = 512
NBLK = CT // BT
EPS = 1e-12


def _sc_gather_body(table_hbm, ids_hbm, out_hbm, idx_v, rows_v, sem):
    wid = lax.axis_index("s") * NC + lax.axis_index("c")
    base = wid * TOK_PER_W
    for f in range(NSUB):
        off = base + f * F
        pltpu.sync_copy(ids_hbm.at[pl.ds(off, F)], idx_v)
        pltpu.async_copy(table_hbm.at[idx_v], rows_v, sem).wait()
        pltpu.sync_copy(rows_v, out_hbm.at[pl.ds(off, F)])


_sc_gather = pl.kernel(
    _sc_gather_body,
    out_type=jax.ShapeDtypeStruct((CT, D), jnp.float32),
    mesh=plsc.VectorSubcoreMesh(
        core_axis_name="c", subcore_axis_name="s", num_cores=NC, num_subcores=NS
    ),
    scratch_types=[
        pltpu.VMEM((F,), jnp.int32),
        pltpu.VMEM((F, D), jnp.float32),
        pltpu.SemaphoreType.DMA,
    ],
)


def _make_tc_ln(chunk, aliased):
    def body(*refs):
        if aliased:
            refs = refs[1:]
        g_ref, tts_ref, eye_ref, pos_ref, wt_ref, gamma_ref, beta_ref, out_ref = refs
        t_row = tts_ref[0]
        t_col = lax.dot_general(
            eye_ref[...], t_row, (((1,), (1,)), ((), ())),
            preferred_element_type=jnp.float32,
        )
        w0 = wt_ref[0:1, :]
        diff = wt_ref[1:2, :] - w0
        x = g_ref[...] + pos_ref[...] + w0 + t_col * diff
        s1 = jnp.sum(x, axis=-1, keepdims=True)
        s2 = jnp.sum(x * x, axis=-1, keepdims=True)
        mean = s1 * (1.0 / D)
        var = jnp.maximum(s2 * (1.0 / D) - mean * mean, 0.0)
        rstd = lax.rsqrt(var + EPS)
        out_ref[...] = (x - mean) * rstd * gamma_ref[...] + beta_ref[...]

    in_specs = [
        pl.BlockSpec((BT, D), lambda i: (i, 0)),
        pl.BlockSpec((1, 1, BT), lambda i: (chunk * NBLK + i, 0, 0)),
        pl.BlockSpec((BT, BT), lambda i: (0, 0)),
        pl.BlockSpec((S, D), lambda i: (0, 0)),
        pl.BlockSpec((2, D), lambda i: (0, 0)),
        pl.BlockSpec((1, D), lambda i: (0, 0)),
        pl.BlockSpec((1, D), lambda i: (0, 0)),
    ]
    kwargs = {}
    if aliased:
        in_specs = [pl.BlockSpec(memory_space=pl.ANY)] + in_specs
        kwargs["input_output_aliases"] = {0: 0}
    return pl.pallas_call(
        body,
        grid=(NBLK,),
        in_specs=in_specs,
        out_specs=pl.BlockSpec((BT, D), lambda i: (chunk * NBLK + i, 0)),
        out_shape=jax.ShapeDtypeStruct((T, D), jnp.float32),
        **kwargs,
    )


def kernel(input_ids, token_type_ids, W_word, W_pos, W_type, gamma, beta):
    ids_flat = input_ids.reshape(T).astype(jnp.int32)
    tts = token_type_ids.reshape(B, 1, S).astype(jnp.float32)
    eye = jnp.eye(BT, dtype=jnp.float32)
    gamma2 = gamma.reshape(1, D)
    beta2 = beta.reshape(1, D)

    gathered = [
        _sc_gather(W_word, ids_flat[c * CT:(c + 1) * CT]) for c in range(NCHUNK)
    ]

    out = None
    for c in range(NCHUNK):
        args = (gathered[c], tts, eye, W_pos, W_type, gamma2, beta2)
        if c == 0:
            out = _make_tc_ln(c, aliased=False)(*args)
        else:
            out = _make_tc_ln(c, aliased=True)(out, *args)

    return out.reshape(B, S, D)
